# Initial kernel scaffold; baseline (speedup 1.0000x reference)
import jax
import jax.numpy as jnp
from jax.experimental import pallas as pl


def kernel(k_c, v_c, fk_c, score_c, K_win, V_win, FK_win, win_score, K_top, V_top, FK_top, heap_score, H_sum, S_sum):
    raise NotImplementedError("write your pallas kernel here")


# speedup vs baseline: 2.7784x; 2.7784x over previous
import functools

import jax
import jax.numpy as jnp
from jax import lax
from jax.experimental import pallas as pl
from jax.experimental.pallas import tpu as pltpu
from jax.experimental.pallas import tpu_sc as plsc

_B, _C, _H, _D, _F, _G = 4, 2048, 16, 128, 128, 2048
_N = _G + _C
_BH = _B * _H
_ROWS = _B * _G * _H
_CHUNK = 128


def _cmp_before(ka, ia, kb, ib):
    return (ka > kb) | ((ka == kb) & (ia < ib))


def _sort_body(score_ref, key_out_ref, gidx_out_ref):
    key = score_ref[...]
    idx = lax.broadcasted_iota(jnp.int32, (_N, _BH), 0)

    i_lin = lax.broadcasted_iota(jnp.int32, (_N, _BH), 0)
    k = 2
    while k <= _N:
        j = k // 2
        while j >= 1:
            if j >= 8:
                m = _N // (2 * j)
                k4 = key.reshape(m, 2, j, _BH)
                i4 = idx.reshape(m, 2, j, _BH)
                ak, bk = k4[:, 0], k4[:, 1]
                ai, bi = i4[:, 0], i4[:, 1]
                r = lax.broadcasted_iota(jnp.int32, (m, 1, 1), 0)
                up = ((r * (2 * j)) & k) == 0
                take = _cmp_before(ak, ai, bk, bi) == up
                nak = jnp.where(take, ak, bk)
                nbk = jnp.where(take, bk, ak)
                nai = jnp.where(take, ai, bi)
                nbi = jnp.where(take, bi, ai)
                key = jnp.stack([nak, nbk], axis=1).reshape(_N, _BH)
                idx = jnp.stack([nai, nbi], axis=1).reshape(_N, _BH)
            else:
                low = (i_lin & j) == 0
                pk = jnp.where(low, pltpu.roll(key, _N - j, 0),
                               pltpu.roll(key, j, 0))
                pi = jnp.where(low, pltpu.roll(idx, _N - j, 0),
                               pltpu.roll(idx, j, 0))
                up = (i_lin & k) == 0
                keep = _cmp_before(key, idx, pk, pi) == (low == up)
                key = jnp.where(keep, key, pk)
                idx = jnp.where(keep, idx, pi)
            j //= 2
        k *= 2

    key_out_ref[...] = key
    col = lax.broadcasted_iota(jnp.int32, (_N, _BH), 1)
    gidx_out_ref[...] = (col // _H) * (_N * _H) + idx * _H + (col % _H)


def _sort_call(score2d):
    return pl.pallas_call(
        _sort_body,
        out_shape=[
            jax.ShapeDtypeStruct((_N, _BH), jnp.float32),
            jax.ShapeDtypeStruct((_N, _BH), jnp.int32),
        ],
    )(score2d)


_NC, _NS = 2, 16
_NW = _NC * _NS
_ROWS_PER_W = _ROWS // _NW
_NCHUNK = _ROWS_PER_W // _CHUNK


def _gather_body(k_hbm, v_hbm, fk_hbm, idx_hbm,
                 ok_hbm, ov_hbm, ofk_hbm,
                 idx_v, bk, bv, bfk, sem):
    wid = lax.axis_index("s") * _NC + lax.axis_index("c")
    base = wid * _ROWS_PER_W

    def step(i, _):
        row0 = base + i * _CHUNK
        pltpu.sync_copy(idx_hbm.at[pl.ds(row0, _CHUNK)], idx_v)
        pltpu.async_copy(k_hbm.at[idx_v], bk, sem).wait()
        pltpu.sync_copy(bk, ok_hbm.at[pl.ds(row0, _CHUNK)])
        pltpu.async_copy(v_hbm.at[idx_v], bv, sem).wait()
        pltpu.sync_copy(bv, ov_hbm.at[pl.ds(row0, _CHUNK)])
        pltpu.async_copy(fk_hbm.at[idx_v], bfk, sem).wait()
        pltpu.sync_copy(bfk, ofk_hbm.at[pl.ds(row0, _CHUNK)])
        return _

    lax.fori_loop(0, _NCHUNK, step, None)


def _gather_call(cat_K, cat_V, cat_FK, idx_flat):
    call = pl.kernel(
        _gather_body,
        mesh=plsc.VectorSubcoreMesh(core_axis_name="c", subcore_axis_name="s"),
        out_type=[jax.ShapeDtypeStruct((_ROWS, _D), jnp.float32)] * 3,
        scratch_types=[
            pltpu.VMEM((_CHUNK,), jnp.int32),
            pltpu.VMEM((_CHUNK, _D), jnp.float32),
            pltpu.VMEM((_CHUNK, _D), jnp.float32),
            pltpu.VMEM((_CHUNK, _D), jnp.float32),
            pltpu.SemaphoreType.DMA,
        ],
    )
    return call(cat_K, cat_V, cat_FK, idx_flat)


def _sums_body(fkt_ref, vt_ref, fkw_ref, vw_ref, fkn_ref, vn_ref,
               hs_ref, ss_ref, ho_ref, so_ref):
    fkt = fkt_ref[0]
    vt = vt_ref[0]
    fkw = fkw_ref[0]
    vw = vw_ref[0]
    fkn = fkn_ref[0]
    vn = vn_ref[0]

    def mm(a, b):
        return lax.dot_general(a, b, (((0,), (0,)), ((), ())),
                               preferred_element_type=jnp.float32)

    ho_ref[0, 0] = hs_ref[0, 0] + mm(fkt, vt) + mm(fkw, vw) - mm(fkn, vn)
    so_ref[0, 0] = (ss_ref[0, 0] + jnp.sum(fkt, axis=0, keepdims=True)
                    + jnp.sum(fkw, axis=0, keepdims=True)
                    - jnp.sum(fkn, axis=0, keepdims=True))


def _sums_call(fkt, vt, fkw, vw, fkn, vn, hs, ss):
    big = pl.BlockSpec((1, _C, _F), lambda b, h: (b, 0, h))
    return pl.pallas_call(
        _sums_body,
        grid=(_B, _H),
        in_specs=[big, big, big, big, big, big,
                  pl.BlockSpec((1, 1, _F, _D), lambda b, h: (b, h, 0, 0)),
                  pl.BlockSpec((1, 1, 1, _F), lambda b, h: (b, h, 0, 0))],
        out_specs=[pl.BlockSpec((1, 1, _F, _D), lambda b, h: (b, h, 0, 0)),
                   pl.BlockSpec((1, 1, 1, _F), lambda b, h: (b, h, 0, 0))],
        out_shape=[
            jax.ShapeDtypeStruct((_B, _H, _F, _D), jnp.float32),
            jax.ShapeDtypeStruct((_B, _H, 1, _F), jnp.float32),
        ],
    )(fkt, vt, fkw, vw, fkn, vn, hs, ss)


def kernel(k_c, v_c, fk_c, score_c, K_win, V_win, FK_win, win_score,
           K_top, V_top, FK_top, heap_score, H_sum, S_sum):
    cat_score = jnp.concatenate([heap_score, win_score], axis=1)
    score2d = cat_score.transpose(1, 0, 2).reshape(_N, _BH)
    skey, gidx = _sort_call(score2d)
    heap_score_new = (skey[:_G].reshape(_G, _B, _H).transpose(1, 0, 2))

    idx_flat = gidx[:_G].reshape(_G, _B, _H).transpose(1, 0, 2).reshape(_ROWS)

    cat_K = jnp.concatenate([K_top, K_win], axis=1).reshape(_B * _N * _H, _D)
    cat_V = jnp.concatenate([V_top, V_win], axis=1).reshape(_B * _N * _H, _D)
    cat_FK = jnp.concatenate([FK_top, FK_win], axis=1).reshape(_B * _N * _H, _F)
    kf, vf, fkf = _gather_call(cat_K, cat_V, cat_FK, idx_flat)
    K_top_new = kf.reshape(_B, _G, _H, _D)
    V_top_new = vf.reshape(_B, _G, _H, _D)
    FK_top_new = fkf.reshape(_B, _G, _H, _F)

    h_new, s_new = _sums_call(
        FK_top.reshape(_B, _G, _H * _F), V_top.reshape(_B, _G, _H * _D),
        FK_win.reshape(_B, _C, _H * _F), V_win.reshape(_B, _C, _H * _D),
        FK_top_new.reshape(_B, _G, _H * _F), V_top_new.reshape(_B, _G, _H * _D),
        H_sum, S_sum.reshape(_B, _H, 1, _F))
    S_sum_new = s_new.reshape(_B, _H, _F)

    return (k_c, v_c, fk_c, score_c,
            K_top_new, V_top_new, FK_top_new, heap_score_new,
            h_new, S_sum_new)
